# baseline (speedup 1.0000x reference)
import functools

import jax
import jax.numpy as jnp
from jax import lax
from jax.experimental import pallas as pl
from jax.experimental.pallas import tpu as pltpu
from jax.experimental.pallas import tpu_sc as plsc

B = 16384
D = 128
L = 16
NC = 2
NS = 16
NW = NC * NS
ROWS_PER_W = B // NW
CHUNK = 128
NCHUNK = ROWS_PER_W // CHUNK
NBUF = 3


def _rsqrt_newton(x):
    i = lax.bitcast_convert_type(x, jnp.int32)
    i = jnp.int32(0x5F3759DF) - lax.shift_right_logical(i, 1)
    y = lax.bitcast_convert_type(i, jnp.float32)
    half_x = jnp.float32(0.5) * x
    for _ in range(3):
        y = y * (jnp.float32(1.5) - half_x * y * y)
    return y


def _make_sc_kernel():
    mesh = plsc.VectorSubcoreMesh(core_axis_name="c", subcore_axis_name="s")

    @functools.partial(
        pl.kernel,
        mesh=mesh,
        compiler_params=pltpu.CompilerParams(needs_layout_passes=False),
        out_type=jax.ShapeDtypeStruct((B,), jnp.float32),
        scratch_types=[
            pltpu.VMEM((NCHUNK, CHUNK), jnp.int32),
            pltpu.VMEM((NBUF, CHUNK, D), jnp.float32),
            pltpu.VMEM((NBUF, CHUNK, D), jnp.float32),
            pltpu.VMEM((ROWS_PER_W,), jnp.float32),
            pltpu.SemaphoreType.DMA,
            pltpu.SemaphoreType.DMA,
            pltpu.SemaphoreType.DMA,
            pltpu.SemaphoreType.DMA,
            pltpu.SemaphoreType.DMA,
            pltpu.SemaphoreType.DMA,
            pltpu.SemaphoreType.DMA,
            pltpu.SemaphoreType.DMA,
        ],
    )
    def sc_kernel(src_hbm, ids_hbm, dst_hbm, table_hbm, out_hbm,
                  idx_v, acc_v, dst_v, out_v,
                  idx_sem, ssem0, ssem1, ssem2, gsem0, gsem1, gsem2,
                  out_sem):
        wid = lax.axis_index("s") * NC + lax.axis_index("c")
        base = wid * ROWS_PER_W
        ssems = (ssem0, ssem1, ssem2)
        gsems = (gsem0, gsem1, gsem2)

        idx_desc = pltpu.async_copy(
            ids_hbm.at[pl.ds(wid * NCHUNK, NCHUNK)], idx_v, idx_sem
        )

        def start_inputs(c, b):
            r0 = base + c * CHUNK
            return (
                pltpu.async_copy(src_hbm.at[pl.ds(r0, CHUNK)], acc_v.at[b],
                                 ssems[b]),
                pltpu.async_copy(dst_hbm.at[pl.ds(r0, CHUNK)], dst_v.at[b],
                                 gsems[b]),
            )

        def start_gather(c, b):
            return pltpu.async_copy(table_hbm.at[idx_v.at[c]], acc_v.at[b],
                                    gsems[b], add=True)

        in_descs = [start_inputs(c, c) for c in range(min(NBUF, NCHUNK))]
        idx_desc.wait()
        gather_descs = []
        for c in range(min(NBUF - 1, NCHUNK)):
            in_descs[c][0].wait()
            gather_descs.append(start_gather(c, c))

        lane = jnp.arange(L, dtype=jnp.int32)
        for c in range(NCHUNK):
            bsel = c % NBUF
            if c + NBUF - 1 < NCHUNK:
                cn = c + NBUF - 1
                in_descs[cn][0].wait()
                gather_descs.append(start_gather(cn, cn % NBUF))
            in_descs[c][1].wait()
            gather_descs[c].wait()
            av, dv = acc_v.at[bsel], dst_v.at[bsel]
            for g in range(CHUNK // L):
                rows = lane + jnp.int32(g * L)

                def body(j, acc):
                    cols = (lane + j) & jnp.int32(D - 1)
                    a = plsc.load_gather(av, [rows, cols])
                    d = plsc.load_gather(dv, [rows, cols])
                    t = a - d
                    return acc + t * t

                acc = lax.fori_loop(0, D, body, jnp.zeros((L,), jnp.float32),
                                    unroll=8)
                x = jnp.maximum(acc, jnp.float32(1e-30))
                out_v[pl.ds(c * CHUNK + g * L, L)] = -(x * _rsqrt_newton(x))
            if c + NBUF < NCHUNK:
                in_descs.append(start_inputs(c + NBUF, bsel))
        pltpu.async_copy(
            out_v, out_hbm.at[pl.ds(base, ROWS_PER_W)], out_sem
        ).wait()

    return sc_kernel


_SC_KERNEL = _make_sc_kernel()


def kernel(src_emb, rel_ids, dst_emb, rel_table):
    ids = rel_ids.astype(jnp.int32).reshape(B // CHUNK, CHUNK)
    return _SC_KERNEL(src_emb, ids, dst_emb, rel_table)


# speedup vs baseline: 1.1486x; 1.1486x over previous
import functools

import jax
import jax.numpy as jnp
from jax import lax
from jax.experimental import pallas as pl
from jax.experimental.pallas import tpu as pltpu
from jax.experimental.pallas import tpu_sc as plsc

B = 16384
D = 128
L = 16
NC = 2
NS = 16
NW = NC * NS
ROWS_PER_W = B // NW
CHUNK = 128
NCHUNK = ROWS_PER_W // CHUNK
NBUF = 3


def _rsqrt_newton(x):
    i = lax.bitcast_convert_type(x, jnp.int32)
    i = jnp.int32(0x5F3759DF) - lax.shift_right_logical(i, 1)
    y = lax.bitcast_convert_type(i, jnp.float32)
    half_x = jnp.float32(0.5) * x
    for _ in range(3):
        y = y * (jnp.float32(1.5) - half_x * y * y)
    return y


def _make_sc_kernel():
    mesh = plsc.VectorSubcoreMesh(core_axis_name="c", subcore_axis_name="s")

    @functools.partial(
        pl.kernel,
        mesh=mesh,
        compiler_params=pltpu.CompilerParams(needs_layout_passes=False),
        out_type=jax.ShapeDtypeStruct((B,), jnp.float32),
        scratch_types=[
            pltpu.VMEM((NCHUNK, CHUNK), jnp.int32),
            pltpu.VMEM((NBUF, CHUNK, D), jnp.float32),
            pltpu.VMEM((NBUF, CHUNK, D), jnp.float32),
            pltpu.VMEM((ROWS_PER_W,), jnp.float32),
            pltpu.SemaphoreType.DMA,
            pltpu.SemaphoreType.DMA,
            pltpu.SemaphoreType.DMA,
            pltpu.SemaphoreType.DMA,
            pltpu.SemaphoreType.DMA,
            pltpu.SemaphoreType.DMA,
            pltpu.SemaphoreType.DMA,
            pltpu.SemaphoreType.DMA,
        ],
    )
    def sc_kernel(src_hbm, ids_hbm, dst_hbm, table_hbm, out_hbm,
                  idx_v, acc_v, dst_v, out_v,
                  idx_sem, ssem0, ssem1, ssem2, gsem0, gsem1, gsem2,
                  out_sem):
        wid = lax.axis_index("s") * NC + lax.axis_index("c")
        base = wid * ROWS_PER_W
        ssems = (ssem0, ssem1, ssem2)
        gsems = (gsem0, gsem1, gsem2)

        idx_desc = pltpu.async_copy(
            ids_hbm.at[pl.ds(wid * NCHUNK, NCHUNK)], idx_v, idx_sem
        )

        def start_inputs(c, b):
            r0 = base + c * CHUNK
            return (
                pltpu.async_copy(src_hbm.at[pl.ds(r0, CHUNK)], acc_v.at[b],
                                 ssems[b]),
                pltpu.async_copy(dst_hbm.at[pl.ds(r0, CHUNK)], dst_v.at[b],
                                 gsems[b]),
            )

        def start_gather(c, b):
            return pltpu.async_copy(table_hbm.at[idx_v.at[c]], acc_v.at[b],
                                    gsems[b], add=True)

        in_descs = [start_inputs(c, c) for c in range(min(NBUF, NCHUNK))]
        idx_desc.wait()
        gather_descs = []
        for c in range(min(NBUF - 1, NCHUNK)):
            in_descs[c][0].wait()
            gather_descs.append(start_gather(c, c))

        lane = jnp.arange(L, dtype=jnp.int32)
        for c in range(NCHUNK):
            bsel = c % NBUF
            if c + NBUF - 1 < NCHUNK:
                cn = c + NBUF - 1
                in_descs[cn][0].wait()
                gather_descs.append(start_gather(cn, cn % NBUF))
            in_descs[c][1].wait()
            gather_descs[c].wait()
            av, dv = acc_v.at[bsel], dst_v.at[bsel]

            def group_body(g, _):
                rows = lane + g * L

                def body(j, acc):
                    cols = (lane + j) & jnp.int32(D - 1)
                    a = plsc.load_gather(av, [rows, cols])
                    d = plsc.load_gather(dv, [rows, cols])
                    t = a - d
                    return acc + t * t

                acc = lax.fori_loop(0, D, body, jnp.zeros((L,), jnp.float32),
                                    unroll=4)
                x = jnp.maximum(acc, jnp.float32(1e-30))
                out_v[pl.ds(c * CHUNK + g * L, L)] = -(x * _rsqrt_newton(x))
                return 0

            lax.fori_loop(0, CHUNK // L, group_body, 0)
            if c + NBUF < NCHUNK:
                in_descs.append(start_inputs(c + NBUF, bsel))
        pltpu.async_copy(
            out_v, out_hbm.at[pl.ds(base, ROWS_PER_W)], out_sem
        ).wait()

    return sc_kernel


_SC_KERNEL = _make_sc_kernel()


def kernel(src_emb, rel_ids, dst_emb, rel_table):
    ids = rel_ids.astype(jnp.int32).reshape(B // CHUNK, CHUNK)
    return _SC_KERNEL(src_emb, ids, dst_emb, rel_table)


# speedup vs baseline: 1.1687x; 1.0175x over previous
import functools

import jax
import jax.numpy as jnp
from jax import lax
from jax.experimental import pallas as pl
from jax.experimental.pallas import tpu as pltpu
from jax.experimental.pallas import tpu_sc as plsc

B = 16384
D = 128
L = 16
NC = 2
NS = 16
NW = NC * NS
ROWS_PER_W = B // NW
CHUNK = 64
NCHUNK = ROWS_PER_W // CHUNK
NBUF = 2


def _rsqrt_newton(x):
    i = lax.bitcast_convert_type(x, jnp.int32)
    i = jnp.int32(0x5F3759DF) - lax.shift_right_logical(i, 1)
    y = lax.bitcast_convert_type(i, jnp.float32)
    half_x = jnp.float32(0.5) * x
    for _ in range(3):
        y = y * (jnp.float32(1.5) - half_x * y * y)
    return y


def _make_sc_kernel():
    mesh = plsc.VectorSubcoreMesh(core_axis_name="c", subcore_axis_name="s")

    @functools.partial(
        pl.kernel,
        mesh=mesh,
        compiler_params=pltpu.CompilerParams(needs_layout_passes=False),
        out_type=jax.ShapeDtypeStruct((B,), jnp.float32),
        scratch_types=[
            pltpu.VMEM((NCHUNK, CHUNK), jnp.int32),
            pltpu.VMEM((NCHUNK, CHUNK, D), jnp.float32),
            pltpu.VMEM((NBUF, CHUNK, D), jnp.float32),
            pltpu.VMEM((NBUF, CHUNK, D), jnp.float32),
            pltpu.VMEM((ROWS_PER_W,), jnp.float32),
            pltpu.SemaphoreType.DMA((NCHUNK,)),
            pltpu.SemaphoreType.DMA((NBUF,)),
            pltpu.SemaphoreType.DMA((NBUF,)),
            pltpu.SemaphoreType.DMA,
            pltpu.SemaphoreType.DMA,
        ],
    )
    def sc_kernel(src_hbm, ids_hbm, dst_hbm, table_hbm, out_hbm,
                  idx_v, rel_v, src_v, dst_v, out_v,
                  gsem, ssem, dsem, idx_sem, out_sem):
        wid = lax.axis_index("s") * NC + lax.axis_index("c")
        base = wid * ROWS_PER_W

        idx_desc = pltpu.async_copy(
            ids_hbm.at[pl.ds(wid * NCHUNK, NCHUNK)], idx_v, idx_sem
        )

        def start_inputs(c, b):
            r0 = base + c * CHUNK
            return (
                pltpu.async_copy(src_hbm.at[pl.ds(r0, CHUNK)], src_v.at[b],
                                 ssem.at[b]),
                pltpu.async_copy(dst_hbm.at[pl.ds(r0, CHUNK)], dst_v.at[b],
                                 dsem.at[b]),
            )

        in_descs = [start_inputs(c, c % NBUF) for c in range(NBUF)]
        idx_desc.wait()
        gather_descs = [
            pltpu.async_copy(table_hbm.at[idx_v.at[c]], rel_v.at[c],
                             gsem.at[c])
            for c in range(NCHUNK)
        ]

        lane = jnp.arange(L, dtype=jnp.int32)
        for c in range(NCHUNK):
            bsel = c % NBUF
            in_descs[c][0].wait()
            in_descs[c][1].wait()
            gather_descs[c].wait()
            sv, dv, rv = src_v.at[bsel], dst_v.at[bsel], rel_v.at[c]

            def group_body(g, _):
                rows = lane + g * L

                def body(j, acc):
                    cols = (lane + j) & jnp.int32(D - 1)
                    s = plsc.load_gather(sv, [rows, cols])
                    r = plsc.load_gather(rv, [rows, cols])
                    d = plsc.load_gather(dv, [rows, cols])
                    t = s + r - d
                    return acc + t * t

                acc = lax.fori_loop(0, D, body, jnp.zeros((L,), jnp.float32),
                                    unroll=4)
                x = jnp.maximum(acc, jnp.float32(1e-30))
                out_v[pl.ds(c * CHUNK + g * L, L)] = -(x * _rsqrt_newton(x))
                return 0

            lax.fori_loop(0, CHUNK // L, group_body, 0)
            if c + NBUF < NCHUNK:
                in_descs.append(start_inputs(c + NBUF, bsel))
        pltpu.async_copy(
            out_v, out_hbm.at[pl.ds(base, ROWS_PER_W)], out_sem
        ).wait()

    return sc_kernel


_SC_KERNEL = _make_sc_kernel()


def kernel(src_emb, rel_ids, dst_emb, rel_table):
    ids = rel_ids.astype(jnp.int32).reshape(B // CHUNK, CHUNK)
    return _SC_KERNEL(src_emb, ids, dst_emb, rel_table)
